# baseline (speedup 1.0000x reference)
import jax
import jax.numpy as jnp
from jax.experimental import pallas as pl
from jax.experimental.pallas import tpu as pltpu

_SIGMA_X = 1.0
_SIGMA_Y = 1.0


def _preprocess_kernel(x_ref, o_ref, sem01, sem3):
    B, _, H, W = x_ref.shape
    cp01 = pltpu.make_async_copy(x_ref.at[:, 0:2], o_ref.at[:, 0:2], sem01)
    cp3 = pltpu.make_async_copy(x_ref.at[:, 3:4], o_ref.at[:, 3:4], sem3)
    cp01.start()
    cp3.start()

    ch2 = x_ref[:, 2, :, :]
    m = (ch2 > 0).astype(jnp.float32)

    ri = jax.lax.broadcasted_iota(jnp.int32, (H, H), 0)
    ci = jax.lax.broadcasted_iota(jnp.int32, (H, H), 1)
    dx = (ri - ci).astype(jnp.float32)
    kx = jnp.exp(-(dx * dx) / (2.0 * _SIGMA_X * _SIGMA_X))
    rj = jax.lax.broadcasted_iota(jnp.int32, (W, W), 0)
    cj = jax.lax.broadcasted_iota(jnp.int32, (W, W), 1)
    dy = (rj - cj).astype(jnp.float32)
    ky = jnp.exp(-(dy * dy) / (2.0 * _SIGMA_Y * _SIGMA_Y))

    t = jnp.concatenate(
        [jnp.dot(kx, m[b], precision=jax.lax.Precision.HIGHEST) for b in range(B)],
        axis=0,
    )
    heat = jnp.dot(t, ky, precision=jax.lax.Precision.HIGHEST)
    heat3 = heat.reshape(B, H, W)
    mx = jnp.max(heat3, axis=(1, 2), keepdims=True)
    normed = heat3 / jnp.where(mx == 0.0, 1.0, mx)

    keep = jnp.sum(m) > 0.0
    o_ref[:, 2, :, :] = jnp.where(keep, normed, ch2)

    cp01.wait()
    cp3.wait()


@jax.jit
def kernel(x):
    return pl.pallas_call(
        _preprocess_kernel,
        out_shape=jax.ShapeDtypeStruct(x.shape, x.dtype),
        scratch_shapes=[pltpu.SemaphoreType.DMA, pltpu.SemaphoreType.DMA],
    )(x)


# speedup vs baseline: 1.0203x; 1.0203x over previous
import jax
import jax.numpy as jnp
from jax.experimental import pallas as pl

_SIGMA_X = 1.0
_SIGMA_Y = 1.0


def _preprocess_kernel(x_ref, o_ref):
    xv = x_ref[...]
    B, _, H, W = xv.shape
    ch2 = xv[:, 2, :, :]
    m = (ch2 > 0).astype(jnp.float32)

    ri = jax.lax.broadcasted_iota(jnp.int32, (H, H), 0)
    ci = jax.lax.broadcasted_iota(jnp.int32, (H, H), 1)
    dx = (ri - ci).astype(jnp.float32)
    kx = jnp.exp(-(dx * dx) / (2.0 * _SIGMA_X * _SIGMA_X))
    rj = jax.lax.broadcasted_iota(jnp.int32, (W, W), 0)
    cj = jax.lax.broadcasted_iota(jnp.int32, (W, W), 1)
    dy = (rj - cj).astype(jnp.float32)
    ky = jnp.exp(-(dy * dy) / (2.0 * _SIGMA_Y * _SIGMA_Y))

    t = jnp.concatenate(
        [jnp.dot(kx, m[b], precision=jax.lax.Precision.HIGHEST) for b in range(B)],
        axis=0,
    )
    heat = jnp.dot(t, ky, precision=jax.lax.Precision.HIGHEST)
    heat3 = heat.reshape(B, H, W)
    mx = jnp.max(heat3, axis=(1, 2), keepdims=True)
    normed = heat3 / jnp.where(mx == 0.0, 1.0, mx)

    keep = jnp.sum(m) > 0.0
    o_ref[...] = xv
    o_ref[:, 2, :, :] = jnp.where(keep, normed, ch2)


@jax.jit
def kernel(x):
    return pl.pallas_call(
        _preprocess_kernel,
        out_shape=jax.ShapeDtypeStruct(x.shape, x.dtype),
    )(x)


# speedup vs baseline: 1.1331x; 1.1106x over previous
import jax
import jax.numpy as jnp
from jax.experimental import pallas as pl

_SIGMA_X = 1.0
_SIGMA_Y = 1.0


def _preprocess_kernel(x_ref, o_ref):
    xv = x_ref[...]
    B, _, H, W = xv.shape
    ch2 = xv[:, 2, :, :]
    m = (ch2 > 0).astype(jnp.float32)

    ri = jax.lax.broadcasted_iota(jnp.int32, (H, H), 0)
    ci = jax.lax.broadcasted_iota(jnp.int32, (H, H), 1)
    dx = (ri - ci).astype(jnp.float32)
    kx = jnp.exp(-(dx * dx) / (2.0 * _SIGMA_X * _SIGMA_X))
    rj = jax.lax.broadcasted_iota(jnp.int32, (W, W), 0)
    cj = jax.lax.broadcasted_iota(jnp.int32, (W, W), 1)
    dy = (rj - cj).astype(jnp.float32)
    ky = jnp.exp(-(dy * dy) / (2.0 * _SIGMA_Y * _SIGMA_Y))

    t = jnp.concatenate(
        [jnp.dot(kx, m[b], precision=jax.lax.Precision.DEFAULT) for b in range(B)],
        axis=0,
    )
    heat = jnp.dot(t, ky, precision=jax.lax.Precision.DEFAULT)
    heat3 = heat.reshape(B, H, W)
    mx = jnp.max(heat3, axis=(1, 2), keepdims=True)
    normed = heat3 / jnp.where(mx == 0.0, 1.0, mx)

    keep = jnp.sum(m) > 0.0
    o_ref[:, 0:2, :, :] = xv[:, 0:2, :, :]
    o_ref[:, 3:4, :, :] = xv[:, 3:4, :, :]
    o_ref[:, 2, :, :] = jnp.where(keep, normed, ch2)


@jax.jit
def kernel(x):
    return pl.pallas_call(
        _preprocess_kernel,
        out_shape=jax.ShapeDtypeStruct(x.shape, x.dtype),
    )(x)
